# baseline (speedup 1.0000x reference)
import functools

import jax
import jax.numpy as jnp
from jax import lax
from jax.experimental import pallas as pl
from jax.experimental.pallas import tpu as pltpu
from jax.experimental.pallas import tpu_sc as plsc

_DIM = 32
_SHAPE_K = 8192
_GAIN_K = 512
_EPS = 1e-05
_TN = 1024
_TK = 1024


def _tc1_body(x_ref, st_ref, gt_ref, si_ref, gi_ref, sc_ref):
    xb = x_ref[...]
    st = st_ref[...]
    d = lax.dot_general(
        xb, st, (((1,), (1,)), ((), ())),
        preferred_element_type=jnp.float32)
    lane = lax.broadcasted_iota(jnp.int32, (128, 128), 1)
    m_parts, i_parts = [], []
    n_strips = _SHAPE_K // 128
    for rb in range(_TN // 128):
        r0 = rb * 128
        acc_m = d[r0:r0 + 128, 0:128]
        acc_c = jnp.zeros((128, 128), jnp.int32)
        for c in range(1, n_strips):
            col = d[r0:r0 + 128, c * 128:(c + 1) * 128]
            upd = col > acc_m
            acc_m = jnp.where(upd, col, acc_m)
            acc_c = jnp.where(upd, jnp.int32(c), acc_c)
        gidx = acc_c * 128 + lane
        m = jnp.max(acc_m, axis=1)
        li = jnp.min(jnp.where(acc_m == m[:, None], gidx, 2 ** 30), axis=1)
        m_parts.append(m)
        i_parts.append(li)
    run_m = jnp.concatenate(m_parts)
    run_i = jnp.concatenate(i_parts)
    g = jnp.log(jnp.clip(run_m, _EPS, None))
    t = gt_ref[0, :]
    lane = lax.broadcasted_iota(jnp.int32, (128, 128), 1)
    gi_parts, gt_parts = [], []
    g_strips = _GAIN_K // 128
    for rb in range(_TN // 128):
        gcol = lax.slice(g, (rb * 128,), (rb * 128 + 128,))[:, None]
        g2 = gcol * gcol
        acc_s = None
        acc_c = jnp.zeros((128, 128), jnp.int32)
        acc_t = None
        for c in range(g_strips):
            ts = lax.slice(t, (c * 128,), (c * 128 + 128,))[None, :]
            p = gcol * ts
            s = (g2 - (p + p)) + ts * ts
            if acc_s is None:
                acc_s, acc_t = s, jnp.broadcast_to(ts, (128, 128))
            else:
                upd = s < acc_s
                acc_s = jnp.where(upd, s, acc_s)
                acc_c = jnp.where(upd, jnp.int32(c), acc_c)
                acc_t = jnp.where(upd, jnp.broadcast_to(ts, (128, 128)), acc_t)
        gidx = acc_c * 128 + lane
        ms = jnp.min(acc_s, axis=1)
        eqm = acc_s == ms[:, None]
        gi_rb = jnp.min(jnp.where(eqm, gidx, 2 ** 30), axis=1)
        tq_rb = jnp.sum(jnp.where(gidx == gi_rb[:, None], acc_t, 0.0), axis=1)
        gi_parts.append(gi_rb)
        gt_parts.append(tq_rb)
    gi = jnp.concatenate(gi_parts)
    gq = jnp.concatenate(gt_parts)
    si_ref[0, 0, :] = run_i
    gi_ref[0, 0, :] = gi
    scale = jnp.exp(gq)
    sc_ref[...] = jnp.broadcast_to(scale[:, None], (_TN, 16))


def _stage1(xf, st, gt2):
    n_blocks = xf.shape[0] // _TN
    return pl.pallas_call(
        _tc1_body,
        grid=(n_blocks,),
        in_specs=[
            pl.BlockSpec((_TN, _DIM), lambda i: (i, 0)),
            pl.BlockSpec((_SHAPE_K, _DIM), lambda i: (0, 0)),
            pl.BlockSpec((1, _GAIN_K), lambda i: (0, 0)),
        ],
        out_specs=[
            pl.BlockSpec((1, 1, _TN), lambda i: (i, 0, 0)),
            pl.BlockSpec((1, 1, _TN), lambda i: (i, 0, 0)),
            pl.BlockSpec((_TN, 16), lambda i: (i, 0)),
        ],
        out_shape=[
            jax.ShapeDtypeStruct((n_blocks, 1, _TN), jnp.int32),
            jax.ShapeDtypeStruct((n_blocks, 1, _TN), jnp.int32),
            jax.ShapeDtypeStruct((n_blocks * _TN, 16), jnp.float32),
        ],
    )(xf, st, gt2)


_LANE = 128


def _make_sc_gather(n_tokens):
    info = plsc.get_sparse_core_info()
    nc, ns = info.num_cores, info.num_subcores
    nw = nc * ns
    chunks_per_w = n_tokens // (nw * _LANE)
    mesh = plsc.VectorSubcoreMesh(core_axis_name="c", subcore_axis_name="s")

    @functools.partial(
        pl.kernel, mesh=mesh,
        out_type=jax.ShapeDtypeStruct((n_tokens // _LANE, _LANE, _LANE),
                                      jnp.float32),
        scratch_types=[
            pltpu.VMEM((chunks_per_w, _LANE), jnp.int32),
            pltpu.VMEM((chunks_per_w * _LANE, 16), jnp.float32),
            pltpu.VMEM((chunks_per_w, _LANE, _LANE), jnp.float32),
            pltpu.SemaphoreType.DMA,
        ],
    )
    def sc_gather(si_hbm, table_hbm, sc_hbm, out_hbm, idx_v, sc_v, rows_v,
                  sem):
        wid = lax.axis_index("s") * nc + lax.axis_index("c")
        base = wid * chunks_per_w
        pltpu.sync_copy(si_hbm.at[pl.ds(base, chunks_per_w)], idx_v)
        pltpu.sync_copy(sc_hbm.at[pl.ds(base * _LANE, chunks_per_w * _LANE)],
                        sc_v)
        copies = [pltpu.async_copy(table_hbm.at[idx_v.at[j]], rows_v.at[j], sem)
                  for j in range(chunks_per_w)]
        for c in copies:
            c.wait()
        for j in range(chunks_per_w):
            for r in range(_LANE):
                sv = sc_v[j * _LANE + r, :]
                a = rows_v[j, r, pl.ds(0, 16)]
                rows_v[j, r, pl.ds(0, 16)] = a * sv
                b = rows_v[j, r, pl.ds(16, 16)]
                rows_v[j, r, pl.ds(16, 16)] = b * sv
            pltpu.sync_copy(rows_v.at[j], out_hbm.at[base + j])

    return sc_gather


def kernel(x, shape_table, gain_table):
    lead = x.shape[:-1]
    xf = x.reshape(-1, x.shape[-1]).astype(jnp.float32)
    n = xf.shape[0]
    gt2 = gain_table.reshape(1, _GAIN_K)
    si3, gi3, sc3 = _stage1(xf, shape_table, gt2)
    shape_ind = si3.reshape(n)
    gain_ind = gi3.reshape(n)
    table_pad = jnp.pad(shape_table, ((0, 0), (0, _LANE - _DIM)))
    rows = _make_sc_gather(n)(shape_ind.reshape(n // _LANE, _LANE),
                              table_pad, sc3)
    quantize = rows.reshape(n, _LANE)[:, :_DIM]
    return (quantize.reshape(*lead, _DIM),
            shape_ind.reshape(lead),
            gain_ind.reshape(lead))


# speedup vs baseline: 1.1832x; 1.1832x over previous
import functools

import jax
import jax.numpy as jnp
from jax import lax
from jax.experimental import pallas as pl
from jax.experimental.pallas import tpu as pltpu
from jax.experimental.pallas import tpu_sc as plsc

_DIM = 32
_SHAPE_K = 8192
_GAIN_K = 512
_EPS = 1e-05
_TN = 1024
_TK = 1024


def _tc1_body(x_ref, st_ref, gt_ref, si_ref, gi_ref, sc_ref):
    xb = x_ref[...]
    st = st_ref[...]
    dT = lax.dot_general(
        st, xb, (((1,), (1,)), ((), ())),
        preferred_element_type=jnp.float32)
    _GRP = 64
    n_grp = _SHAPE_K // _GRP
    a_iota = lax.broadcasted_iota(jnp.int32, (_GRP, 128), 0)
    m_parts, i_parts = [], []
    for ts in range(_TN // 128):
        t0 = ts * 128
        acc_m = dT[0:_GRP, t0:t0 + 128]
        acc_g = jnp.zeros((_GRP, 128), jnp.int32)
        for gg in range(1, n_grp):
            blk = dT[gg * _GRP:(gg + 1) * _GRP, t0:t0 + 128]
            upd = blk > acc_m
            acc_m = jnp.where(upd, blk, acc_m)
            acc_g = jnp.where(upd, jnp.int32(gg), acc_g)
        vm = acc_m
        vi = acc_g * _GRP + a_iota
        h = _GRP // 2
        while h >= 1:
            vm_lo, vm_hi = vm[0:h, :], vm[h:2 * h, :]
            vi_lo, vi_hi = vi[0:h, :], vi[h:2 * h, :]
            take_hi = (vm_hi > vm_lo) | ((vm_hi == vm_lo) & (vi_hi < vi_lo))
            vm = jnp.where(take_hi, vm_hi, vm_lo)
            vi = jnp.where(take_hi, vi_hi, vi_lo)
            h //= 2
        m_parts.append(vm[0])
        i_parts.append(vi[0])
    run_m = jnp.concatenate(m_parts)
    run_i = jnp.concatenate(i_parts)
    g = jnp.log(jnp.clip(run_m, _EPS, None))
    t = gt_ref[0, :]
    lane = lax.broadcasted_iota(jnp.int32, (128, 128), 1)
    gi_parts, gt_parts = [], []
    g_strips = _GAIN_K // 128
    for rb in range(_TN // 128):
        gcol = lax.slice(g, (rb * 128,), (rb * 128 + 128,))[:, None]
        g2 = gcol * gcol
        acc_s = None
        acc_c = jnp.zeros((128, 128), jnp.int32)
        acc_t = None
        for c in range(g_strips):
            ts = lax.slice(t, (c * 128,), (c * 128 + 128,))[None, :]
            p = gcol * ts
            s = (g2 - (p + p)) + ts * ts
            if acc_s is None:
                acc_s, acc_t = s, jnp.broadcast_to(ts, (128, 128))
            else:
                upd = s < acc_s
                acc_s = jnp.where(upd, s, acc_s)
                acc_c = jnp.where(upd, jnp.int32(c), acc_c)
                acc_t = jnp.where(upd, jnp.broadcast_to(ts, (128, 128)), acc_t)
        gidx = acc_c * 128 + lane
        ms = jnp.min(acc_s, axis=1)
        eqm = acc_s == ms[:, None]
        gi_rb = jnp.min(jnp.where(eqm, gidx, 2 ** 30), axis=1)
        tq_rb = jnp.sum(jnp.where(gidx == gi_rb[:, None], acc_t, 0.0), axis=1)
        gi_parts.append(gi_rb)
        gt_parts.append(tq_rb)
    gi = jnp.concatenate(gi_parts)
    gq = jnp.concatenate(gt_parts)
    si_ref[0, 0, :] = run_i
    gi_ref[0, 0, :] = gi
    scale = jnp.exp(gq)
    sc_ref[...] = jnp.broadcast_to(scale[:, None], (_TN, 16))


def _stage1(xf, st, gt2):
    n_blocks = xf.shape[0] // _TN
    return pl.pallas_call(
        _tc1_body,
        grid=(n_blocks,),
        in_specs=[
            pl.BlockSpec((_TN, _DIM), lambda i: (i, 0)),
            pl.BlockSpec((_SHAPE_K, _DIM), lambda i: (0, 0)),
            pl.BlockSpec((1, _GAIN_K), lambda i: (0, 0)),
        ],
        out_specs=[
            pl.BlockSpec((1, 1, _TN), lambda i: (i, 0, 0)),
            pl.BlockSpec((1, 1, _TN), lambda i: (i, 0, 0)),
            pl.BlockSpec((_TN, 16), lambda i: (i, 0)),
        ],
        out_shape=[
            jax.ShapeDtypeStruct((n_blocks, 1, _TN), jnp.int32),
            jax.ShapeDtypeStruct((n_blocks, 1, _TN), jnp.int32),
            jax.ShapeDtypeStruct((n_blocks * _TN, 16), jnp.float32),
        ],
    )(xf, st, gt2)


_LANE = 128


def _make_sc_gather(n_tokens):
    info = plsc.get_sparse_core_info()
    nc, ns = info.num_cores, info.num_subcores
    nw = nc * ns
    chunks_per_w = n_tokens // (nw * _LANE)
    mesh = plsc.VectorSubcoreMesh(core_axis_name="c", subcore_axis_name="s")

    @functools.partial(
        pl.kernel, mesh=mesh,
        out_type=jax.ShapeDtypeStruct((n_tokens // _LANE, _LANE, _LANE),
                                      jnp.float32),
        scratch_types=[
            pltpu.VMEM((chunks_per_w, _LANE), jnp.int32),
            pltpu.VMEM((chunks_per_w * _LANE, 16), jnp.float32),
            pltpu.VMEM((chunks_per_w, _LANE, _LANE), jnp.float32),
            pltpu.SemaphoreType.DMA,
        ],
    )
    def sc_gather(si_hbm, table_hbm, sc_hbm, out_hbm, idx_v, sc_v, rows_v,
                  sem):
        wid = lax.axis_index("s") * nc + lax.axis_index("c")
        base = wid * chunks_per_w
        pltpu.sync_copy(si_hbm.at[pl.ds(base, chunks_per_w)], idx_v)
        pltpu.sync_copy(sc_hbm.at[pl.ds(base * _LANE, chunks_per_w * _LANE)],
                        sc_v)
        copies = [pltpu.async_copy(table_hbm.at[idx_v.at[j]], rows_v.at[j], sem)
                  for j in range(chunks_per_w)]
        for c in copies:
            c.wait()
        for j in range(chunks_per_w):
            for r in range(_LANE):
                sv = sc_v[j * _LANE + r, :]
                a = rows_v[j, r, pl.ds(0, 16)]
                rows_v[j, r, pl.ds(0, 16)] = a * sv
                b = rows_v[j, r, pl.ds(16, 16)]
                rows_v[j, r, pl.ds(16, 16)] = b * sv
            pltpu.sync_copy(rows_v.at[j], out_hbm.at[base + j])

    return sc_gather


def kernel(x, shape_table, gain_table):
    lead = x.shape[:-1]
    xf = x.reshape(-1, x.shape[-1]).astype(jnp.float32)
    n = xf.shape[0]
    gt2 = gain_table.reshape(1, _GAIN_K)
    si3, gi3, sc3 = _stage1(xf, shape_table, gt2)
    shape_ind = si3.reshape(n)
    gain_ind = gi3.reshape(n)
    table_pad = jnp.pad(shape_table, ((0, 0), (0, _LANE - _DIM)))
    rows = _make_sc_gather(n)(shape_ind.reshape(n // _LANE, _LANE),
                              table_pad, sc3)
    quantize = rows.reshape(n, _LANE)[:, :_DIM]
    return (quantize.reshape(*lead, _DIM),
            shape_ind.reshape(lead),
            gain_ind.reshape(lead))


# speedup vs baseline: 1.3151x; 1.1115x over previous
import functools

import jax
import jax.numpy as jnp
from jax import lax
from jax.experimental import pallas as pl
from jax.experimental.pallas import tpu as pltpu
from jax.experimental.pallas import tpu_sc as plsc

_DIM = 32
_SHAPE_K = 8192
_GAIN_K = 512
_EPS = 1e-05
_TN = 1024
_TK = 1024


def _tc1_body(x_ref, st_ref, gt_ref, si_ref, gi_ref, sc_ref):
    xb = x_ref[...]
    st = st_ref[...]
    dT = lax.dot_general(
        st, xb, (((1,), (1,)), ((), ())),
        preferred_element_type=jnp.float32)
    _GRP = 64
    n_grp = _SHAPE_K // _GRP
    a_iota = lax.broadcasted_iota(jnp.int32, (_GRP, 128), 0)
    m_parts, i_parts = [], []
    for ts in range(_TN // 128):
        t0 = ts * 128
        acc_m = dT[0:_GRP, t0:t0 + 128]
        acc_g = jnp.zeros((_GRP, 128), jnp.int32)
        for gg in range(1, n_grp):
            blk = dT[gg * _GRP:(gg + 1) * _GRP, t0:t0 + 128]
            upd = blk > acc_m
            acc_m = jnp.where(upd, blk, acc_m)
            acc_g = jnp.where(upd, jnp.int32(gg), acc_g)
        vm = acc_m
        vi = acc_g * _GRP + a_iota
        h = _GRP // 2
        while h >= 1:
            vm_lo, vm_hi = vm[0:h, :], vm[h:2 * h, :]
            vi_lo, vi_hi = vi[0:h, :], vi[h:2 * h, :]
            take_hi = (vm_hi > vm_lo) | ((vm_hi == vm_lo) & (vi_hi < vi_lo))
            vm = jnp.where(take_hi, vm_hi, vm_lo)
            vi = jnp.where(take_hi, vi_hi, vi_lo)
            h //= 2
        m_parts.append(vm[0])
        i_parts.append(vi[0])
    run_m = jnp.concatenate(m_parts)
    run_i = jnp.concatenate(i_parts)
    g = jnp.log(jnp.clip(run_m, _EPS, None))
    tcol = gt_ref[...]
    n_ggrp = _GAIN_K // _GRP
    tb = [jnp.broadcast_to(tcol[gg * _GRP:(gg + 1) * _GRP], (_GRP, 128))
          for gg in range(n_ggrp)]
    tsqb = [b * b for b in tb]
    ga_iota = lax.broadcasted_iota(jnp.int32, (_GRP, 128), 0)
    gi_parts, gt_parts = [], []
    for ts in range(_TN // 128):
        t0 = ts * 128
        grow = lax.slice(g, (t0,), (t0 + 128,))[None, :]
        gb = jnp.broadcast_to(grow, (_GRP, 128))
        g2b = gb * gb
        acc_s = acc_t = None
        acc_g = jnp.zeros((_GRP, 128), jnp.int32)
        for gg in range(n_ggrp):
            p = gb * tb[gg]
            s = (g2b - (p + p)) + tsqb[gg]
            if acc_s is None:
                acc_s, acc_t = s, tb[gg]
            else:
                upd = s < acc_s
                acc_s = jnp.where(upd, s, acc_s)
                acc_g = jnp.where(upd, jnp.int32(gg), acc_g)
                acc_t = jnp.where(upd, tb[gg], acc_t)
        vm, vt = acc_s, acc_t
        vi = acc_g * _GRP + ga_iota
        h = _GRP // 2
        while h >= 1:
            vm_lo, vm_hi = vm[0:h, :], vm[h:2 * h, :]
            vi_lo, vi_hi = vi[0:h, :], vi[h:2 * h, :]
            vt_lo, vt_hi = vt[0:h, :], vt[h:2 * h, :]
            take_hi = (vm_hi < vm_lo) | ((vm_hi == vm_lo) & (vi_hi < vi_lo))
            vm = jnp.where(take_hi, vm_hi, vm_lo)
            vi = jnp.where(take_hi, vi_hi, vi_lo)
            vt = jnp.where(take_hi, vt_hi, vt_lo)
            h //= 2
        gi_parts.append(vi[0])
        gt_parts.append(vt[0])
    gi = jnp.concatenate(gi_parts)
    gq = jnp.concatenate(gt_parts)
    si_ref[0, 0, :] = run_i
    gi_ref[0, 0, :] = gi
    scale = jnp.exp(gq)
    sc_ref[...] = jnp.broadcast_to(scale[:, None], (_TN, 16))


def _stage1(xf, st, gt2):
    n_blocks = xf.shape[0] // _TN
    return pl.pallas_call(
        _tc1_body,
        grid=(n_blocks,),
        in_specs=[
            pl.BlockSpec((_TN, _DIM), lambda i: (i, 0)),
            pl.BlockSpec((_SHAPE_K, _DIM), lambda i: (0, 0)),
            pl.BlockSpec((_GAIN_K, 1), lambda i: (0, 0)),
        ],
        out_specs=[
            pl.BlockSpec((1, 1, _TN), lambda i: (i, 0, 0)),
            pl.BlockSpec((1, 1, _TN), lambda i: (i, 0, 0)),
            pl.BlockSpec((_TN, 16), lambda i: (i, 0)),
        ],
        out_shape=[
            jax.ShapeDtypeStruct((n_blocks, 1, _TN), jnp.int32),
            jax.ShapeDtypeStruct((n_blocks, 1, _TN), jnp.int32),
            jax.ShapeDtypeStruct((n_blocks * _TN, 16), jnp.float32),
        ],
    )(xf, st, gt2)


_LANE = 128


def _make_sc_gather(n_tokens):
    info = plsc.get_sparse_core_info()
    nc, ns = info.num_cores, info.num_subcores
    nw = nc * ns
    chunks_per_w = n_tokens // (nw * _LANE)
    mesh = plsc.VectorSubcoreMesh(core_axis_name="c", subcore_axis_name="s")

    @functools.partial(
        pl.kernel, mesh=mesh,
        out_type=jax.ShapeDtypeStruct((n_tokens // _LANE, _LANE, _LANE),
                                      jnp.float32),
        scratch_types=[
            pltpu.VMEM((chunks_per_w, _LANE), jnp.int32),
            pltpu.VMEM((chunks_per_w * _LANE, 16), jnp.float32),
            pltpu.VMEM((chunks_per_w, _LANE, _LANE), jnp.float32),
            pltpu.SemaphoreType.DMA,
        ],
    )
    def sc_gather(si_hbm, table_hbm, sc_hbm, out_hbm, idx_v, sc_v, rows_v,
                  sem):
        wid = lax.axis_index("s") * nc + lax.axis_index("c")
        base = wid * chunks_per_w
        pltpu.sync_copy(si_hbm.at[pl.ds(base, chunks_per_w)], idx_v)
        pltpu.sync_copy(sc_hbm.at[pl.ds(base * _LANE, chunks_per_w * _LANE)],
                        sc_v)
        copies = [pltpu.async_copy(table_hbm.at[idx_v.at[j]], rows_v.at[j], sem)
                  for j in range(chunks_per_w)]
        for c in copies:
            c.wait()
        for j in range(chunks_per_w):
            for r in range(_LANE):
                sv = sc_v[j * _LANE + r, :]
                a = rows_v[j, r, pl.ds(0, 16)]
                rows_v[j, r, pl.ds(0, 16)] = a * sv
                b = rows_v[j, r, pl.ds(16, 16)]
                rows_v[j, r, pl.ds(16, 16)] = b * sv
            pltpu.sync_copy(rows_v.at[j], out_hbm.at[base + j])

    return sc_gather


def kernel(x, shape_table, gain_table):
    lead = x.shape[:-1]
    xf = x.reshape(-1, x.shape[-1]).astype(jnp.float32)
    n = xf.shape[0]
    gt2 = gain_table.reshape(_GAIN_K, 1)
    si3, gi3, sc3 = _stage1(xf, shape_table, gt2)
    shape_ind = si3.reshape(n)
    gain_ind = gi3.reshape(n)
    table_pad = jnp.pad(shape_table, ((0, 0), (0, _LANE - _DIM)))
    rows = _make_sc_gather(n)(shape_ind.reshape(n // _LANE, _LANE),
                              table_pad, sc3)
    quantize = rows.reshape(n, _LANE)[:, :_DIM]
    return (quantize.reshape(*lead, _DIM),
            shape_ind.reshape(lead),
            gain_ind.reshape(lead))
